# baseline (speedup 1.0000x reference)
import functools

import jax
import jax.numpy as jnp
from jax import lax
from jax.experimental import pallas as pl
from jax.experimental.pallas import tpu as pltpu
from jax.experimental.pallas import tpu_sc as plsc

_NUM_CORES = 2
_NUM_SUBCORES = 16
_NW = _NUM_CORES * _NUM_SUBCORES

_CHUNK = 128
_CPB = 4
_BLOCK = _CHUNK * _CPB
_NBUF = 3


@functools.partial(jax.jit, static_argnames=("n_rows", "dim"))
def _sc_gather(table, idx, *, n_rows, dim):
    b_per_w = n_rows // _NW
    nblocks = b_per_w // _BLOCK

    mesh = plsc.VectorSubcoreMesh(
        core_axis_name="c", subcore_axis_name="s", num_cores=_NUM_CORES
    )

    @functools.partial(
        pl.kernel,
        mesh=mesh,
        compiler_params=pltpu.CompilerParams(use_tc_tiling_on_sc=False),
        out_type=jax.ShapeDtypeStruct((n_rows, dim), table.dtype),
        scratch_types=(
            [pltpu.VMEM((b_per_w,), jnp.int32)]
            + [pltpu.VMEM((_BLOCK, dim), table.dtype) for _ in range(_NBUF)]
            + [pltpu.SemaphoreType.DMA for _ in range(_NBUF)]
            + [pltpu.SemaphoreType.DMA for _ in range(_NBUF)]
        ),
    )
    def run(table_hbm, idx_hbm, out_hbm, idx_v, *rest):
        bufs = rest[:_NBUF]
        gsems = rest[_NBUF : 2 * _NBUF]
        osems = rest[2 * _NBUF :]

        wid = lax.axis_index("s") * _NUM_CORES + lax.axis_index("c")
        base = wid * b_per_w
        pltpu.sync_copy(idx_hbm.at[pl.ds(base, b_per_w)], idx_v)

        def _mo(v):
            return v if isinstance(v, int) else pl.multiple_of(v, 8)

        def fire_gathers(o, b):
            for j in range(_CPB):
                off = _mo(o * _BLOCK + j * _CHUNK)
                pltpu.make_async_copy(
                    table_hbm.at[idx_v.at[pl.ds(off, _CHUNK)]],
                    bufs[b].at[pl.ds(j * _CHUNK, _CHUNK)],
                    gsems[b],
                ).start()

        def wait_gathers(o, b):
            for j in range(_CPB):
                off = _mo(o * _BLOCK + j * _CHUNK)
                pltpu.make_async_copy(
                    table_hbm.at[idx_v.at[pl.ds(off, _CHUNK)]],
                    bufs[b].at[pl.ds(j * _CHUNK, _CHUNK)],
                    gsems[b],
                ).wait()

        def out_copy(o, b):
            return pltpu.make_async_copy(
                bufs[b],
                out_hbm.at[pl.ds(_mo(base + o * _BLOCK), _BLOCK)],
                osems[b],
            )

        fire_gathers(0, 0)
        fire_gathers(1, 1)

        def block(o, carry):
            b = lax.rem(o, _NBUF)
            for bb in range(_NBUF):

                @pl.when(b == bb)
                def _():
                    wait_gathers(o, bb)
                    out_copy(o, bb).start()
                    @pl.when(o >= 1)
                    def _():
                        out_copy(o - 1, (bb + _NBUF - 1) % _NBUF).wait()

                    @pl.when(o + 2 < nblocks)
                    def _():
                        fire_gathers(o + 2, (bb + 2) % _NBUF)

            return carry

        lax.fori_loop(0, nblocks, block, 0)
        out_copy(nblocks - 1, (nblocks - 1) % _NBUF).wait()

    return run(table, idx)


def kernel(x, lengths, table):
    del lengths
    batch, hist = x.shape
    dim = table.shape[1]
    idx = x.reshape(-1).astype(jnp.int32)
    out = _sc_gather(table, idx, n_rows=batch * hist, dim=dim)
    return out.reshape(batch, hist, dim)
